# Initial kernel scaffold; baseline (speedup 1.0000x reference)
import jax
import jax.numpy as jnp
from jax.experimental import pallas as pl


def kernel(x, H, params):
    raise NotImplementedError("write your pallas kernel here")


# speedup vs baseline: 1.7217x; 1.7217x over previous
import functools

import jax
import jax.numpy as jnp
from jax.experimental import pallas as pl

_IN = 256
_HID = 128
_NEG_SLOPE_ATT = 0.2
_NEG_SLOPE_MLP = 0.01


def _lrelu(v, slope):
    return jnp.where(v > 0, v, slope * v)


def _ln(v, g, b):
    mu = jnp.mean(v, axis=-1, keepdims=True)
    var = jnp.mean(jnp.square(v - mu), axis=-1, keepdims=True)
    return (v - mu) * jax.lax.rsqrt(var + 1e-5) * g + b


def _dot(a, b):
    return jax.lax.dot_general(a, b, (((1,), (0,)), ((), ())),
                               preferred_element_type=jnp.float32)


def _dot_t(a, b):
    return jax.lax.dot_general(a, b, (((0,), (0,)), ((), ())),
                               preferred_element_type=jnp.float32)


def _dot_rr(a, b):
    return jax.lax.dot_general(a, b, (((1,), (1,)), ((), ())),
                               preferred_element_type=jnp.float32)


def _p1_kernel(x_ref, H_ref, W_ref, W2_ref, W3_ref, ahi_ref, wc_ref,
               alo_ref, a2lo_ref, a2hi_ref,
               edge_ref, es_ref, xs_ref, M_ref, me_ref, *, heads):
    x = x_ref[...]
    Hm = H_ref[...]
    for h in range(heads):
        W = W_ref[h]
        W2 = W2_ref[h]
        W3 = W3_ref[h]
        xt = _dot(x, W)
        x4 = _dot(x, W2)
        c = jnp.sum(wc_ref[h] * alo_ref[h])
        s = _dot(x4, ahi_ref[h]) + c
        e1 = _lrelu(s, _NEG_SLOPE_ATT)
        w = jnp.exp(e1 - jnp.max(e1))
        num = _dot(Hm, w * xt)
        den = _dot(Hm, w)
        mean_xt = jnp.mean(xt, axis=0, keepdims=True)
        edge = jnp.where(den > 0, num / jnp.where(den > 0, den, 1.0), mean_xt)
        e4 = _dot(edge, W3)
        es = _dot(e4, a2hi_ref[h])
        xs = _dot_rr(a2lo_ref[h], x4)
        M = _lrelu(jnp.max(es) + jnp.max(xs), _NEG_SLOPE_ATT)
        edge_ref[h] = edge
        es_ref[h] = es
        xs_ref[h] = xs
        M_ref[h] = jnp.full((1, 1), M, jnp.float32)
        me_ref[h] = jnp.mean(edge, axis=0, keepdims=True)


def _p2_kernel(H_ref, x_ref, edge_ref, es_ref, xs_ref, M_ref, me_ref,
               hmW_ref, hmb_ref, lng_ref, lnb_ref,
               fW1_ref, fb1_ref, fW2_ref, fb2_ref, flng_ref, flnb_ref,
               out_ref, *, heads):
    Hb = H_ref[...]
    xb = x_ref[...]
    hs = []
    for h in range(heads):
        es = es_ref[h]
        xs = xs_ref[h]
        M = M_ref[h][0, 0]
        T = Hb * jnp.exp(_lrelu(es + xs, _NEG_SLOPE_ATT) - M)
        den = jnp.sum(T, axis=0, keepdims=True)
        num = _dot_t(T, edge_ref[h])
        dcol = den.T
        node = jnp.where(dcol > 0, num / jnp.where(dcol > 0, dcol, 1.0),
                         me_ref[h])
        hs.append(jnp.where(node > 0, node, jnp.exp(node) - 1.0))
    hcat = jnp.concatenate(hs, axis=-1)
    x1 = _lrelu(_dot(hcat, hmW_ref[...]) + hmb_ref[...], _NEG_SLOPE_MLP) + xb
    x1 = _ln(x1, lng_ref[...], lnb_ref[...])
    f = _lrelu(_dot(x1, fW1_ref[...]) + fb1_ref[...], _NEG_SLOPE_MLP)
    f = _lrelu(_dot(f, fW2_ref[...]) + fb2_ref[...], _NEG_SLOPE_MLP)
    f = _ln(f, flng_ref[...], flnb_ref[...])
    x2 = _ln(f + x1, lng_ref[...], lnb_ref[...])
    out_ref[...] = x2 + xb


def _layer(xb, Hm, bp, *, nb):
    n_nodes, n_in = xb.shape
    n_edges = Hm.shape[0]
    heads = len(bp['heads'])
    hid = bp['heads'][0]['W'].shape[1]

    W_s = jnp.stack([hp['W'] for hp in bp['heads']])
    W2_s = jnp.stack([hp['W2'] for hp in bp['heads']])
    W3_s = jnp.stack([hp['W3'] for hp in bp['heads']])
    ahi_s = jnp.stack([hp['a'][hid:] for hp in bp['heads']])
    wc_s = jnp.stack([hp['wc'][None, :] for hp in bp['heads']])
    alo_s = jnp.stack([hp['a'][:hid, 0][None, :] for hp in bp['heads']])
    a2lo_s = jnp.stack([hp['a2'][:hid, 0][None, :] for hp in bp['heads']])
    a2hi_s = jnp.stack([hp['a2'][hid:] for hp in bp['heads']])

    f32 = jnp.float32
    edge_s, es_s, xs_s, M_s, me_s = pl.pallas_call(
        functools.partial(_p1_kernel, heads=heads),
        out_shape=(
            jax.ShapeDtypeStruct((heads, n_edges, hid), f32),
            jax.ShapeDtypeStruct((heads, n_edges, 1), f32),
            jax.ShapeDtypeStruct((heads, 1, n_nodes), f32),
            jax.ShapeDtypeStruct((heads, 1, 1), f32),
            jax.ShapeDtypeStruct((heads, 1, hid), f32),
        ),
    )(xb, Hm, W_s, W2_s, W3_s, ahi_s, wc_s, alo_s, a2lo_s, a2hi_s)

    grid = (n_nodes // nb,)
    full = lambda *shape: pl.BlockSpec(shape, lambda j: (0,) * len(shape))
    out = pl.pallas_call(
        functools.partial(_p2_kernel, heads=heads),
        grid=grid,
        in_specs=[
            pl.BlockSpec((n_edges, nb), lambda j: (0, j)),
            pl.BlockSpec((nb, n_in), lambda j: (j, 0)),
            full(heads, n_edges, hid),
            full(heads, n_edges, 1),
            pl.BlockSpec((heads, 1, nb), lambda j: (0, 0, j)),
            full(heads, 1, 1),
            full(heads, 1, hid),
            full(n_in, n_in),
            full(1, n_in), full(1, n_in), full(1, n_in),
            full(n_in, n_in), full(1, n_in),
            full(n_in, n_in), full(1, n_in),
            full(1, n_in), full(1, n_in),
        ],
        out_specs=pl.BlockSpec((nb, n_in), lambda j: (j, 0)),
        out_shape=jax.ShapeDtypeStruct((n_nodes, n_in), f32),
    )(Hm, xb, edge_s, es_s, xs_s, M_s, me_s,
      bp['hm_W'], bp['hm_b'][None, :], bp['ln_g'][None, :],
      bp['ln_b'][None, :], bp['ffn_W1'], bp['ffn_b1'][None, :],
      bp['ffn_W2'], bp['ffn_b2'][None, :], bp['ffn_ln_g'][None, :],
      bp['ffn_ln_b'][None, :])
    return out


def kernel(x, H, params):
    xb = x[0]
    Hm = H[0]
    for bp in params:
        xb = _layer(xb, Hm, bp, nb=512)
    return xb[None]
